# baseline (speedup 1.0000x reference)
import jax
import jax.numpy as jnp
from jax import lax
from jax.experimental import pallas as pl
from jax.experimental.pallas import tpu as pltpu
from jax.experimental.pallas import tpu_sc as plsc

_D = 512


def _body(l1_hbm, l0_hbm, out_hbm):
    cid = lax.axis_index("c")

    @pl.when(cid == 0)
    def _():
        pltpu.sync_copy(l1_hbm.at[pl.ds(0, 64)], out_hbm.at[pl.ds(0, 64)])

    @pl.when(cid == 1)
    def _():
        pltpu.sync_copy(l0_hbm.at[pl.ds(0, 64), 0], out_hbm.at[pl.ds(64, 64)])


@jax.jit
def kernel(layer1, layer0):
    mesh = plsc.ScalarSubcoreMesh(axis_name="c", num_cores=2)
    f = pl.kernel(
        _body,
        out_type=jax.ShapeDtypeStruct((128, 2, _D), jnp.float32),
        mesh=mesh,
    )
    l1_p = layer1.reshape(8192, 2, _D)
    l0_q = layer0.reshape(4096, 2, 2, _D)
    return f(l1_p, l0_q).reshape(256, _D)
